# baseline (speedup 1.0000x reference)
import jax
import jax.numpy as jnp
from jax.experimental import pallas as pl
from jax.experimental.pallas import tpu as pltpu


def _gcn_body(a_ref, seq_ref, w_ref, adj_t_ref, adj_b_ref, bias_ref,
              out_ref):
    a = a_ref[0]

    def one(adj_ref, half):
        tmp = jax.lax.dot_general(
            adj_ref[...], seq_ref[...],
            dimension_numbers=(((1,), (0,)), ((), ())),
            preferred_element_type=jnp.float32)
        acc = jax.lax.dot_general(
            tmp, w_ref[...],
            dimension_numbers=(((1,), (1,)), ((), ())),
            preferred_element_type=jnp.float32)
        acc = acc + bias_ref[...]
        out_ref[half, :, :] = jnp.where(acc >= 0, acc, a * acc)

    one(adj_t_ref, 0)
    one(adj_b_ref, 1)


def _block_m(n: int, cap: int) -> int:
    best = 8
    for bm in range(8, cap + 1, 8):
        if n % bm == 0:
            best = bm
    return best


def kernel(seq, adj, W, bias, prelu_a):
    b, n, d_in = seq.shape
    d_out = W.shape[0]
    m = b * n
    seq2 = seq.reshape(m, d_in)
    adj2 = adj.reshape(m, n)
    bias2 = bias.reshape(1, d_out)
    a2 = jnp.asarray(prelu_a, jnp.float32).reshape(1)

    half = m // 2
    bm = _block_m(half, 256)
    nsteps = half // bm
    grid = (nsteps,)

    out = pl.pallas_call(
        _gcn_body,
        grid=grid,
        in_specs=[
            pl.BlockSpec(memory_space=pltpu.SMEM),
            pl.BlockSpec((n, d_in), lambda i: (0, 0)),
            pl.BlockSpec((d_out, d_in), lambda i: (0, 0)),
            pl.BlockSpec((bm, n), lambda i: (i, 0)),
            pl.BlockSpec((bm, n), lambda i, o=nsteps: (i + o, 0)),
            pl.BlockSpec((1, d_out), lambda i: (0, 0)),
        ],
        out_specs=pl.BlockSpec((2, bm, d_out), lambda i: (0, i, 0)),
        out_shape=jax.ShapeDtypeStruct((2, half, d_out), jnp.float32),
        compiler_params=pltpu.CompilerParams(
            dimension_semantics=("arbitrary",)),
    )(a2, seq2, W, adj2, adj2, bias2)
    return out.reshape(b, n, d_out)


# speedup vs baseline: 1.0926x; 1.0926x over previous
import jax
import jax.numpy as jnp
from jax.experimental import pallas as pl
from jax.experimental.pallas import tpu as pltpu


def _gcn_body(a_ref, seq_ref, w_ref, adj_ref, bias_ref, out_ref):
    tmp = jax.lax.dot_general(
        adj_ref[...], seq_ref[...],
        dimension_numbers=(((1,), (0,)), ((), ())),
        preferred_element_type=jnp.float32)
    acc = jax.lax.dot_general(
        tmp, w_ref[...],
        dimension_numbers=(((1,), (1,)), ((), ())),
        preferred_element_type=jnp.float32)
    acc = acc + bias_ref[...]
    a = a_ref[0]
    out_ref[...] = jnp.where(acc >= 0, acc, a * acc)


def _block_m(n: int) -> int:
    best = 8
    for bm in range(8, 513, 8):
        if n % bm == 0:
            best = bm
    return best


def kernel(seq, adj, W, bias, prelu_a):
    b, n, d_in = seq.shape
    d_out = W.shape[0]
    m = b * n
    seq2 = seq.reshape(m, d_in)
    adj2 = adj.reshape(m, n)
    bias2 = bias.reshape(1, d_out)
    a2 = jnp.asarray(prelu_a, jnp.float32).reshape(1)

    bm = _block_m(m)
    grid = (m // bm,)

    out = pl.pallas_call(
        _gcn_body,
        grid=grid,
        in_specs=[
            pl.BlockSpec(memory_space=pltpu.SMEM),
            pl.BlockSpec((n, d_in), lambda i: (0, 0)),
            pl.BlockSpec((d_out, d_in), lambda i: (0, 0)),
            pl.BlockSpec((bm, n), lambda i: (i, 0)),
            pl.BlockSpec((1, d_out), lambda i: (0, 0)),
        ],
        out_specs=pl.BlockSpec((bm, d_out), lambda i: (i, 0)),
        out_shape=jax.ShapeDtypeStruct((m, d_out), jnp.float32),
        compiler_params=pltpu.CompilerParams(
            dimension_semantics=("arbitrary",)),
    )(a2, seq2, W, adj2, bias2)
    return out.reshape(b, n, d_out)
